# Initial kernel scaffold; baseline (speedup 1.0000x reference)
import jax
import jax.numpy as jnp
from jax.experimental import pallas as pl


def kernel(x, edge_index, edge_attr, batch, params):
    raise NotImplementedError("write your pallas kernel here")


# speedup vs baseline: 7.5722x; 7.5722x over previous
import functools

import jax
import jax.numpy as jnp
from jax import lax
from jax.experimental import pallas as pl
from jax.experimental.pallas import tpu as pltpu
from jax.experimental.pallas import tpu_sc as plsc

_N = 10000
_E = 320000
_D = 128
_NG = 256
_NPAD = 10112
_NC, _NS = 2, 16
_NT = _NC * _NS
_EPT = _E // _NT
_CH = 128
_NFULL = _EPT // _CH
_TAIL = _EPT - _NFULL * _CH
_RPC = _NPAD // _NS
_BR = _NPAD // 16

_HIGH = lax.Precision.HIGHEST


def _dot(a, b):
    return jnp.dot(a, b, preferred_element_type=jnp.float32, precision=_HIGH)


def _bond_table(b0, b1, b2):
    i4096 = lax.broadcasted_iota(jnp.int32, (4096, 1), 0)
    i16 = lax.broadcasted_iota(jnp.int32, (1, 16), 1)
    oh0 = ((i4096 // 256) == i16).astype(jnp.float32)
    oh1 = (((i4096 // 16) % 16) == i16).astype(jnp.float32)
    oh2 = ((i4096 % 16) == i16).astype(jnp.float32)
    return _dot(oh0, b0) + _dot(oh1, b1) + _dot(oh2, b2)


def _encode_body(x_ref, aemb_ref, b00_ref, b01_ref, b02_ref,
                 b10_ref, b11_ref, b12_ref, h_ref, t0_ref, t1_ref):
    i = pl.program_id(0)
    i64 = lax.broadcasted_iota(jnp.int32, (1, 64), 1)
    h = None
    for f in range(9):
        oh = (x_ref[:, f : f + 1] == i64).astype(jnp.float32)
        part = _dot(oh, aemb_ref[f])
        h = part if h is None else h + part
    h_ref[...] = h

    @pl.when(i == 0)
    def _():
        t0_ref[...] = _bond_table(b00_ref[...], b01_ref[...], b02_ref[...])
        t1_ref[...] = _bond_table(b10_ref[...], b11_ref[...], b12_ref[...])


def _mlp_body(relu_out, h_ref, ag_ref, eps_ref, w1_ref, b1_ref, g1_ref,
              be1_ref, w2_ref, b2_ref, go_ref, bo_ref, hout_ref,
              hh1_ref, hh2_ref, st_ref):
    ph = pl.program_id(0)
    i = pl.program_id(1)
    valid = (lax.broadcasted_iota(jnp.int32, (_BR, 1), 0) + i * _BR) < _N

    @pl.when(ph == 0)
    def _():
        pre = h_ref[...] * (1.0 + eps_ref[0, 0]) + ag_ref[0] + ag_ref[1]
        hh = _dot(pre, w1_ref[...]) + b1_ref[...]
        hh = jnp.where(valid, hh, 0.0)
        hh1_ref[pl.ds(i * _BR, _BR), :] = hh

        @pl.when(i == 0)
        def _():
            st_ref[...] = jnp.zeros((8, 2 * _D), jnp.float32)
        st_ref[0:1, :] = st_ref[0:1, :] + jnp.sum(hh, axis=0, keepdims=True)
        st_ref[1:2, :] = st_ref[1:2, :] + jnp.sum(hh * hh, axis=0,
                                                  keepdims=True)

    @pl.when(ph == 1)
    def _():
        m = st_ref[0:1, :] / _N
        v = st_ref[1:2, :] / _N - m * m
        hh = hh1_ref[pl.ds(i * _BR, _BR), :]
        hh = (hh - m) / jnp.sqrt(v + 1e-5) * g1_ref[...] + be1_ref[...]
        hh = jnp.maximum(hh, 0.0)
        hh = _dot(hh, w2_ref[...]) + b2_ref[...]
        hh = jnp.where(valid, hh, 0.0)
        hh2_ref[pl.ds(i * _BR, _BR), :] = hh

        @pl.when(i == 0)
        def _():
            st_ref[2:4, :] = jnp.zeros((2, 2 * _D), jnp.float32)
        st_ref[2:3, 0:_D] = (st_ref[2:3, 0:_D]
                             + jnp.sum(hh, axis=0, keepdims=True))
        st_ref[3:4, 0:_D] = (st_ref[3:4, 0:_D]
                             + jnp.sum(hh * hh, axis=0, keepdims=True))

    @pl.when(ph == 2)
    def _():
        m = st_ref[2:3, 0:_D] / _N
        v = st_ref[3:4, 0:_D] / _N - m * m
        hh = hh2_ref[pl.ds(i * _BR, _BR), :]
        hh = (hh - m) / jnp.sqrt(v + 1e-5) * go_ref[...] + bo_ref[...]
        if relu_out:
            hh = jnp.maximum(hh, 0.0)
        hout_ref[...] = hh + h_ref[...]


def _head_body(h_ref, batch_ref, wcat_ref, bvec_ref, p_ref, r_ref):
    i = pl.program_id(0)
    valid = (lax.broadcasted_iota(jnp.int32, (_BR, 1), 0) + i * _BR) < _N
    h = h_ref[...]
    p = _dot(h, wcat_ref[...]) + bvec_ref[...]
    nk = 1.0 / (1.0 + jnp.exp(-p[:, 0:1]))
    lane = lax.broadcasted_iota(jnp.int32, (_BR, _D), 1)
    nk_b = jnp.broadcast_to(nk, (_BR, _D))
    p_ref[...] = jnp.where(lane == 0, nk_b, p)
    i256 = lax.broadcasted_iota(jnp.int32, (_NG, 1), 0)
    ot = (i256 == batch_ref[0]).astype(jnp.float32)
    m = jnp.where(lane == 0, nk_b, jnp.where(lane == 1, 1.0 - nk_b, 0.0))
    m = jnp.where(valid, m, 0.0)
    rpart = _dot(ot, m)

    @pl.when(i == 0)
    def _():
        r_ref[...] = rpart + 1e-8

    @pl.when(i > 0)
    def _():
        r_ref[...] = r_ref[...] + rpart


def _fin_body(pk_ref, pc_ref, out_ref):
    k = jnp.sum(pk_ref[...], axis=0, keepdims=True)
    c = jnp.sum(pc_ref[...], axis=0, keepdims=True)
    out_ref[0:1, :] = k + 1e-8
    out_ref[1:2, :] = (c - k) + 1e-8


def _msg_body(h_hbm, t_hbm, row_hbm, col_hbm, pidx_hbm, aggr_hbm,
              rowv, colv, pidxv, hrowv, browv,
              trowv, tcolv, tpidxv, throwv, tbrowv,
              sem1, sem2, aggr_sh):
    cid = lax.axis_index("c")
    sid = lax.axis_index("s")
    wid = cid * _NS + sid
    zero = jnp.zeros((16,), jnp.float32)

    def zrow(i, _):
        for j in range(_D // 16):
            hrowv[i, pl.ds(j * 16, 16)] = zero
        return 0
    lax.fori_loop(0, _CH, zrow, 0)
    rbase = sid * _RPC
    for k in range(_RPC // _CH):
        pltpu.sync_copy(hrowv, aggr_sh.at[pl.ds(rbase + k * _CH, _CH)])
    rem = _RPC - (_RPC // _CH) * _CH
    if rem:
        pltpu.sync_copy(hrowv.at[pl.ds(0, rem)],
                        aggr_sh.at[pl.ds(rbase + (_RPC // _CH) * _CH, rem)])
    plsc.subcore_barrier()

    ebase = wid * _EPT

    def chunk(j, _):
        off = ebase + j * _CH
        pltpu.sync_copy(row_hbm.at[pl.ds(off, _CH)], rowv)
        pltpu.sync_copy(col_hbm.at[pl.ds(off, _CH)], colv)
        pltpu.sync_copy(pidx_hbm.at[pl.ds(off, _CH)], pidxv)
        c1 = pltpu.async_copy(h_hbm.at[rowv], hrowv, sem1)
        c2 = pltpu.async_copy(t_hbm.at[pidxv], browv, sem2)
        c1.wait()
        c2.wait()

        def rowop(i, _):
            for jj in range(_D // 16):
                s = pl.ds(jj * 16, 16)
                hrowv[i, s] = jnp.maximum(hrowv[i, s] + browv[i, s], 0.0)
            return 0
        lax.fori_loop(0, _CH, rowop, 0)
        pltpu.sync_copy(hrowv, aggr_sh.at[colv], add=True)
        return 0
    lax.fori_loop(0, _NFULL, chunk, 0)

    offt = ebase + _NFULL * _CH
    pltpu.sync_copy(row_hbm.at[pl.ds(offt, _TAIL)], trowv)
    pltpu.sync_copy(col_hbm.at[pl.ds(offt, _TAIL)], tcolv)
    pltpu.sync_copy(pidx_hbm.at[pl.ds(offt, _TAIL)], tpidxv)
    c1 = pltpu.async_copy(h_hbm.at[trowv], throwv, sem1)
    c2 = pltpu.async_copy(t_hbm.at[tpidxv], tbrowv, sem2)
    c1.wait()
    c2.wait()

    def rowopt(i, _):
        for jj in range(_D // 16):
            s = pl.ds(jj * 16, 16)
            throwv[i, s] = jnp.maximum(throwv[i, s] + tbrowv[i, s], 0.0)
        return 0
    lax.fori_loop(0, _TAIL, rowopt, 0)
    pltpu.sync_copy(throwv, aggr_sh.at[tcolv], add=True)

    plsc.subcore_barrier()

    obase = cid * _NPAD + rbase
    for k in range(_RPC // _CH):
        pltpu.sync_copy(aggr_sh.at[pl.ds(rbase + k * _CH, _CH)], hrowv)
        pltpu.sync_copy(hrowv, aggr_hbm.at[pl.ds(obase + k * _CH, _CH)])
    if rem:
        pltpu.sync_copy(aggr_sh.at[pl.ds(rbase + (_RPC // _CH) * _CH, rem)],
                        hrowv.at[pl.ds(0, rem)])
        pltpu.sync_copy(hrowv.at[pl.ds(0, rem)],
                        aggr_hbm.at[pl.ds(obase + (_RPC // _CH) * _CH, rem)])


def _ehead_body(a_hbm, c_hbm, batch_hbm, row_hbm, col_hbm,
                ek_hbm, pk_hbm, pc_hbm,
                rowv, colv, av, cv, sgv, ekv,
                trowv, tcolv, tav, tcv, tsgv, tekv,
                acck, accc, foldv, sem1, sem2, sem3):
    cid = lax.axis_index("c")
    sid = lax.axis_index("s")
    wid = cid * _NS + sid
    zero = jnp.zeros((16,), jnp.float32)
    ones = jnp.ones((16,), jnp.float32)
    lanes = lax.iota(jnp.int32, 16)

    def zacc(i, _):
        acck[pl.ds(i * 16, 16)] = zero
        accc[pl.ds(i * 16, 16)] = zero
        return 0
    lax.fori_loop(0, 16 * _NG // 16, zacc, 0)

    ebase = wid * _EPT

    def do_group(src_a, src_c, src_sg, dst_ek, g):
        z = src_a[pl.ds(g * 16, 16)] + src_c[pl.ds(g * 16, 16)]
        ek = 1.0 / (1.0 + jnp.exp(-z))
        dst_ek[pl.ds(g * 16, 16)] = ek
        idx = lanes * _NG + src_sg[pl.ds(g * 16, 16)]
        plsc.addupdate_scatter(acck, [idx], ek)
        plsc.addupdate_scatter(accc, [idx], ones)

    def chunk(j, _):
        off = ebase + j * _CH
        pltpu.sync_copy(row_hbm.at[pl.ds(off, _CH)], rowv)
        pltpu.sync_copy(col_hbm.at[pl.ds(off, _CH)], colv)
        c1 = pltpu.async_copy(a_hbm.at[rowv], av, sem1)
        c2 = pltpu.async_copy(c_hbm.at[colv], cv, sem2)
        c3 = pltpu.async_copy(batch_hbm.at[rowv], sgv, sem3)
        c1.wait()
        c2.wait()
        c3.wait()
        for g in range(_CH // 16):
            do_group(av, cv, sgv, ekv, g)
        pltpu.sync_copy(ekv, ek_hbm.at[pl.ds(off, _CH)])
        return 0
    lax.fori_loop(0, _NFULL, chunk, 0)

    offt = ebase + _NFULL * _CH
    pltpu.sync_copy(row_hbm.at[pl.ds(offt, _TAIL)], trowv)
    pltpu.sync_copy(col_hbm.at[pl.ds(offt, _TAIL)], tcolv)
    c1 = pltpu.async_copy(a_hbm.at[trowv], tav, sem1)
    c2 = pltpu.async_copy(c_hbm.at[tcolv], tcv, sem2)
    c3 = pltpu.async_copy(batch_hbm.at[trowv], tsgv, sem3)
    c1.wait()
    c2.wait()
    c3.wait()
    do_group(tav, tcv, tsgv, tekv, 0)
    pltpu.sync_copy(tekv, ek_hbm.at[pl.ds(offt, _TAIL)])

    for acc, dst in ((acck, pk_hbm), (accc, pc_hbm)):
        def foldop(v, _):
            s = zero
            for l in range(16):
                s = s + acc[pl.ds(l * _NG + v * 16, 16)]
            foldv[pl.ds(v * 16, 16)] = s
            return 0
        lax.fori_loop(0, _NG // 16, foldop, 0)
        pltpu.sync_copy(foldv, dst.at[pl.ds(wid * _NG, _NG)])


@functools.lru_cache(maxsize=None)
def _sc_kernels():
    mesh = plsc.VectorSubcoreMesh(core_axis_name="c", subcore_axis_name="s",
                                  num_cores=_NC, num_subcores=_NS)
    msg = pl.kernel(
        _msg_body,
        out_type=jax.ShapeDtypeStruct((_NC * _NPAD, _D), jnp.float32),
        mesh=mesh,
        scratch_types=[
            pltpu.VMEM((_CH,), jnp.int32),
            pltpu.VMEM((_CH,), jnp.int32),
            pltpu.VMEM((_CH,), jnp.int32),
            pltpu.VMEM((_CH, _D), jnp.float32),
            pltpu.VMEM((_CH, _D), jnp.float32),
            pltpu.VMEM((_TAIL,), jnp.int32),
            pltpu.VMEM((_TAIL,), jnp.int32),
            pltpu.VMEM((_TAIL,), jnp.int32),
            pltpu.VMEM((_TAIL, _D), jnp.float32),
            pltpu.VMEM((_TAIL, _D), jnp.float32),
            pltpu.SemaphoreType.DMA,
            pltpu.SemaphoreType.DMA,
            pltpu.VMEM_SHARED((_NPAD, _D), jnp.float32),
        ],
    )

    ehead = pl.kernel(
        _ehead_body,
        out_type=(
            jax.ShapeDtypeStruct((_E,), jnp.float32),
            jax.ShapeDtypeStruct((_NT * _NG,), jnp.float32),
            jax.ShapeDtypeStruct((_NT * _NG,), jnp.float32),
        ),
        mesh=mesh,
        compiler_params=pltpu.CompilerParams(needs_layout_passes=False),
        scratch_types=[
            pltpu.VMEM((_CH,), jnp.int32),
            pltpu.VMEM((_CH,), jnp.int32),
            pltpu.VMEM((_CH,), jnp.float32),
            pltpu.VMEM((_CH,), jnp.float32),
            pltpu.VMEM((_CH,), jnp.int32),
            pltpu.VMEM((_CH,), jnp.float32),
            pltpu.VMEM((_TAIL,), jnp.int32),
            pltpu.VMEM((_TAIL,), jnp.int32),
            pltpu.VMEM((_TAIL,), jnp.float32),
            pltpu.VMEM((_TAIL,), jnp.float32),
            pltpu.VMEM((_TAIL,), jnp.int32),
            pltpu.VMEM((_TAIL,), jnp.float32),
            pltpu.VMEM((16 * _NG,), jnp.float32),
            pltpu.VMEM((16 * _NG,), jnp.float32),
            pltpu.VMEM((_NG,), jnp.float32),
            pltpu.SemaphoreType.DMA,
            pltpu.SemaphoreType.DMA,
            pltpu.SemaphoreType.DMA,
        ],
    )
    return msg, ehead

def _full(shape):
    nd = len(shape)
    return pl.BlockSpec(shape, lambda *ids: (0,) * nd)


def _build_tc(interpret=False):
    encode = pl.pallas_call(
        _encode_body,
        grid=(16,),
        in_specs=[
            pl.BlockSpec((_BR, 9), lambda i: (i, 0)),
            _full((9, 64, _D)),
            _full((16, _D)), _full((16, _D)), _full((16, _D)),
            _full((16, _D)), _full((16, _D)), _full((16, _D)),
        ],
        out_specs=(
            pl.BlockSpec((_BR, _D), lambda i: (i, 0)),
            _full((4096, _D)),
            _full((4096, _D)),
        ),
        out_shape=(
            jax.ShapeDtypeStruct((_NPAD, _D), jnp.float32),
            jax.ShapeDtypeStruct((4096, _D), jnp.float32),
            jax.ShapeDtypeStruct((4096, _D), jnp.float32),
        ),
        interpret=interpret,
    )

    def mlp(relu_out):
        return pl.pallas_call(
            functools.partial(_mlp_body, relu_out),
            grid=(3, 16),
            in_specs=[
                pl.BlockSpec((_BR, _D), lambda p, i: (i, 0)),
                pl.BlockSpec((2, _BR, _D), lambda p, i: (0, i, 0)),
                _full((1, 1)),
                _full((_D, 2 * _D)), _full((1, 2 * _D)),
                _full((1, 2 * _D)), _full((1, 2 * _D)),
                _full((2 * _D, _D)), _full((1, _D)),
                _full((1, _D)), _full((1, _D)),
            ],
            out_specs=pl.BlockSpec((_BR, _D), lambda p, i: (i, 0)),
            out_shape=jax.ShapeDtypeStruct((_NPAD, _D), jnp.float32),
            scratch_shapes=[
                pltpu.VMEM((_NPAD, 2 * _D), jnp.float32),
                pltpu.VMEM((_NPAD, _D), jnp.float32),
                pltpu.VMEM((8, 2 * _D), jnp.float32),
            ],
            interpret=interpret,
        )

    head = pl.pallas_call(
        _head_body,
        grid=(16,),
        in_specs=[
            pl.BlockSpec((_BR, _D), lambda i: (i, 0)),
            pl.BlockSpec((1, 1, _BR), lambda i: (i, 0, 0)),
            _full((_D, _D)),
            _full((1, _D)),
        ],
        out_specs=(
            pl.BlockSpec((_BR, _D), lambda i: (i, 0)),
            _full((_NG, _D)),
        ),
        out_shape=(
            jax.ShapeDtypeStruct((_NPAD, _D), jnp.float32),
            jax.ShapeDtypeStruct((_NG, _D), jnp.float32),
        ),
        interpret=interpret,
    )

    fin = pl.pallas_call(
        _fin_body,
        out_shape=jax.ShapeDtypeStruct((2, _NG), jnp.float32),
        interpret=interpret,
    )
    return encode, mlp(True), mlp(False), head, fin


_encode, _mlp0, _mlp1, _head, _fin = _build_tc()


def kernel(x, edge_index, edge_attr, batch, params):
    row = edge_index[0]
    col = edge_index[1]
    pidx = edge_attr[:, 0] * 256 + edge_attr[:, 1] * 16 + edge_attr[:, 2]
    l0, l1 = params['layers']
    _msg, _ehead = _sc_kernels()

    h0, t0, t1 = _encode(x, params['atom_emb'],
                         l0['bond_emb'][0], l0['bond_emb'][1],
                         l0['bond_emb'][2],
                         l1['bond_emb'][0], l1['bond_emb'][1],
                         l1['bond_emb'][2])
    ag0 = _msg(h0, t0, row, col, pidx)
    eps0 = l0['eps'].reshape(1, 1)
    h1 = _mlp0(h0, ag0.reshape(_NC, _NPAD, _D), eps0,
               l0['W1'], l0['b1'].reshape(1, -1),
               l0['bn1_g'].reshape(1, -1), l0['bn1_b'].reshape(1, -1),
               l0['W2'], l0['b2'].reshape(1, -1),
               l0['bno_g'].reshape(1, -1), l0['bno_b'].reshape(1, -1))
    ag1 = _msg(h1, t1, row, col, pidx)
    eps1 = l1['eps'].reshape(1, 1)
    h2 = _mlp1(h1, ag1.reshape(_NC, _NPAD, _D), eps1,
               l1['W1'], l1['b1'].reshape(1, -1),
               l1['bn1_g'].reshape(1, -1), l1['bn1_b'].reshape(1, -1),
               l1['W2'], l1['b2'].reshape(1, -1),
               l1['bno_g'].reshape(1, -1), l1['bno_b'].reshape(1, -1))

    wcat = jnp.zeros((_D, _D), jnp.float32)
    wcat = wcat.at[:, 0].set(params['node_att_W'][:, 0])
    wcat = wcat.at[:, 1].set(params['edge_att_W'][:_D, 0])
    wcat = wcat.at[:, 2].set(params['edge_att_W'][_D:, 0])
    bvec = jnp.zeros((1, _D), jnp.float32)
    bvec = bvec.at[0, 0].set(params['node_att_b'][0])
    bvec = bvec.at[0, 1].set(params['edge_att_b'][0])

    batch_pad = jnp.concatenate(
        [batch, jnp.zeros((_NPAD - _N,), jnp.int32)]).reshape(16, 1, _BR)
    p, r = _head(h2, batch_pad, wcat, bvec)
    p = p[:_N]
    a = p[:, 1]
    c = p[:, 2]
    ek, pk, pc = _ehead(a, c, batch, row, col)
    fin = _fin(pk.reshape(_NT, _NG), pc.reshape(_NT, _NG))

    node_key = p[:, 0:1]
    edge_key = ek.reshape(_E, 1)
    node_key_num = r[:, 0:1]
    node_env_num = r[:, 1:2]
    edge_key_num = fin[0].reshape(_NG, 1)
    edge_env_num = fin[1].reshape(_NG, 1)
    return (node_key, edge_key, node_key_num, node_env_num,
            edge_key_num, edge_env_num)


# speedup vs baseline: 10.3654x; 1.3689x over previous
import functools

import jax
import jax.numpy as jnp
from jax import lax
from jax.experimental import pallas as pl
from jax.experimental.pallas import tpu as pltpu
from jax.experimental.pallas import tpu_sc as plsc

_N = 10000
_E = 320000
_D = 128
_NG = 256
_NPAD = 10112
_NC, _NS = 2, 16
_NT = _NC * _NS
_EPT = _E // _NT
_CH = 128
_MCH = 80
_NCH = _E // _MCH
_CPT = _NCH // _NT
_NFULL = _EPT // _CH
_TAIL = _EPT - _NFULL * _CH
_RPC = _NPAD // _NS
_BR = _NPAD // 16

_HIGH = lax.Precision.HIGHEST


def _dot(a, b):
    return jnp.dot(a, b, preferred_element_type=jnp.float32, precision=_HIGH)


def _bond_table(b0, b1, b2):
    i4096 = lax.broadcasted_iota(jnp.int32, (4096, 1), 0)
    i16 = lax.broadcasted_iota(jnp.int32, (1, 16), 1)
    oh0 = ((i4096 // 256) == i16).astype(jnp.float32)
    oh1 = (((i4096 // 16) % 16) == i16).astype(jnp.float32)
    oh2 = ((i4096 % 16) == i16).astype(jnp.float32)
    return _dot(oh0, b0) + _dot(oh1, b1) + _dot(oh2, b2)


def _encode_body(x_ref, aemb_ref, b00_ref, b01_ref, b02_ref,
                 b10_ref, b11_ref, b12_ref, h_ref, t0_ref, t1_ref):
    i = pl.program_id(0)
    i64 = lax.broadcasted_iota(jnp.int32, (1, 64), 1)
    h = None
    for f in range(9):
        oh = (x_ref[:, f : f + 1] == i64).astype(jnp.float32)
        part = _dot(oh, aemb_ref[f])
        h = part if h is None else h + part
    h_ref[...] = h

    @pl.when(i == 0)
    def _():
        t0_ref[...] = _bond_table(b00_ref[...], b01_ref[...], b02_ref[...])
        t1_ref[...] = _bond_table(b10_ref[...], b11_ref[...], b12_ref[...])


def _mlp_body(relu_out, h_ref, ag_ref, eps_ref, w1_ref, b1_ref, g1_ref,
              be1_ref, w2_ref, b2_ref, go_ref, bo_ref, hout_ref,
              hh1_ref, hh2_ref, st_ref):
    ph = pl.program_id(0)
    i = pl.program_id(1)
    valid = (lax.broadcasted_iota(jnp.int32, (_BR, 1), 0) + i * _BR) < _N

    @pl.when(ph == 0)
    def _():
        pre = h_ref[...] * (1.0 + eps_ref[0, 0]) + ag_ref[0] + ag_ref[1]
        hh = _dot(pre, w1_ref[...]) + b1_ref[...]
        hh = jnp.where(valid, hh, 0.0)
        hh1_ref[pl.ds(i * _BR, _BR), :] = hh

        @pl.when(i == 0)
        def _():
            st_ref[...] = jnp.zeros((8, 2 * _D), jnp.float32)
        st_ref[0:1, :] = st_ref[0:1, :] + jnp.sum(hh, axis=0, keepdims=True)
        st_ref[1:2, :] = st_ref[1:2, :] + jnp.sum(hh * hh, axis=0,
                                                  keepdims=True)

    @pl.when(ph == 1)
    def _():
        m = st_ref[0:1, :] / _N
        v = st_ref[1:2, :] / _N - m * m
        hh = hh1_ref[pl.ds(i * _BR, _BR), :]
        hh = (hh - m) / jnp.sqrt(v + 1e-5) * g1_ref[...] + be1_ref[...]
        hh = jnp.maximum(hh, 0.0)
        hh = _dot(hh, w2_ref[...]) + b2_ref[...]
        hh = jnp.where(valid, hh, 0.0)
        hh2_ref[pl.ds(i * _BR, _BR), :] = hh

        @pl.when(i == 0)
        def _():
            st_ref[2:4, :] = jnp.zeros((2, 2 * _D), jnp.float32)
        st_ref[2:3, 0:_D] = (st_ref[2:3, 0:_D]
                             + jnp.sum(hh, axis=0, keepdims=True))
        st_ref[3:4, 0:_D] = (st_ref[3:4, 0:_D]
                             + jnp.sum(hh * hh, axis=0, keepdims=True))

    @pl.when(ph == 2)
    def _():
        m = st_ref[2:3, 0:_D] / _N
        v = st_ref[3:4, 0:_D] / _N - m * m
        hh = hh2_ref[pl.ds(i * _BR, _BR), :]
        hh = (hh - m) / jnp.sqrt(v + 1e-5) * go_ref[...] + bo_ref[...]
        if relu_out:
            hh = jnp.maximum(hh, 0.0)
        hout_ref[...] = hh + h_ref[...]


def _head_body(h_ref, batch_ref, wcat_ref, bvec_ref, p_ref, r_ref):
    i = pl.program_id(0)
    valid = (lax.broadcasted_iota(jnp.int32, (_BR, 1), 0) + i * _BR) < _N
    h = h_ref[...]
    p = _dot(h, wcat_ref[...]) + bvec_ref[...]
    nk = 1.0 / (1.0 + jnp.exp(-p[:, 0:1]))
    lane = lax.broadcasted_iota(jnp.int32, (_BR, _D), 1)
    nk_b = jnp.broadcast_to(nk, (_BR, _D))
    p_ref[...] = jnp.where(lane == 0, nk_b, p)
    i256 = lax.broadcasted_iota(jnp.int32, (_NG, 1), 0)
    ot = (i256 == batch_ref[0]).astype(jnp.float32)
    m = jnp.where(lane == 0, nk_b, jnp.where(lane == 1, 1.0 - nk_b, 0.0))
    m = jnp.where(valid, m, 0.0)
    rpart = _dot(ot, m)

    @pl.when(i == 0)
    def _():
        r_ref[...] = rpart + 1e-8

    @pl.when(i > 0)
    def _():
        r_ref[...] = r_ref[...] + rpart


def _fin_body(pk_ref, pc_ref, out_ref):
    k = jnp.sum(pk_ref[...], axis=0, keepdims=True)
    c = jnp.sum(pc_ref[...], axis=0, keepdims=True)
    out_ref[0:1, :] = k + 1e-8
    out_ref[1:2, :] = (c - k) + 1e-8


def _relu_add(hv, bv):
    def rowop(i, _):
        for jj in range(_D // 16):
            s = pl.ds(jj * 16, 16)
            hv[i, s] = jnp.maximum(hv[i, s] + bv[i, s], 0.0)
        return 0
    lax.fori_loop(0, _MCH, rowop, 0)


def _msg_body(h_hbm, t_hbm, pk_hbm, aggr_hbm,
              idx0, hrow0, brow0, idx1, hrow1, brow1,
              semh0, semt0, semh1, semt1, aggr_sh):
    cid = lax.axis_index("c")
    sid = lax.axis_index("s")
    wid = cid * _NS + sid
    zero = jnp.zeros((16,), jnp.float32)

    def zrow(i, _):
        for j in range(_D // 16):
            hrow0[i, pl.ds(j * 16, 16)] = zero
        return 0
    lax.fori_loop(0, _MCH, zrow, 0)
    rbase = sid * _RPC
    for k in range(_RPC // _MCH):
        pltpu.sync_copy(hrow0, aggr_sh.at[pl.ds(rbase + k * _MCH, _MCH)])
    rem = _RPC - (_RPC // _MCH) * _MCH
    if rem:
        pltpu.sync_copy(hrow0.at[pl.ds(0, rem)],
                        aggr_sh.at[pl.ds(rbase + (_RPC // _MCH) * _MCH, rem)])
    plsc.subcore_barrier()

    bufs = ((idx0, hrow0, brow0, semh0, semt0),
            (idx1, hrow1, brow1, semh1, semt1))
    cbase = wid * _CPT

    for b in range(2):
        idxb, hv, bv, sh, st_ = bufs[b]
        pltpu.sync_copy(pk_hbm.at[cbase + b], idxb)
        pltpu.async_copy(h_hbm.at[idxb.at[0]], hv, sh)
        pltpu.async_copy(t_hbm.at[idxb.at[2]], bv, st_)

    def pairbody(g, _):
        for b in range(2):
            idxb, hv, bv, sh, st_ = bufs[b]
            c = g * 2 + b
            pltpu.make_async_copy(h_hbm.at[idxb.at[0]], hv, sh).wait()
            pltpu.make_async_copy(t_hbm.at[idxb.at[2]], bv, st_).wait()
            _relu_add(hv, bv)
            pltpu.sync_copy(hv, aggr_sh.at[idxb.at[1]], add=True)

            @pl.when(c + 2 < _CPT)
            def _():
                pltpu.sync_copy(pk_hbm.at[cbase + c + 2], idxb)
                pltpu.async_copy(h_hbm.at[idxb.at[0]], hv, sh)
                pltpu.async_copy(t_hbm.at[idxb.at[2]], bv, st_)
        return 0
    lax.fori_loop(0, _CPT // 2, pairbody, 0)

    idxb, hv, bv, sh, st_ = bufs[0]
    pltpu.make_async_copy(h_hbm.at[idxb.at[0]], hv, sh).wait()
    pltpu.make_async_copy(t_hbm.at[idxb.at[2]], bv, st_).wait()
    _relu_add(hv, bv)
    pltpu.sync_copy(hv, aggr_sh.at[idxb.at[1]], add=True)

    plsc.subcore_barrier()

    obase = cid * _NPAD + rbase
    for k in range(_RPC // _MCH):
        pltpu.sync_copy(aggr_sh.at[pl.ds(rbase + k * _MCH, _MCH)], hrow0)
        pltpu.sync_copy(hrow0, aggr_hbm.at[pl.ds(obase + k * _MCH, _MCH)])
    if rem:
        pltpu.sync_copy(aggr_sh.at[pl.ds(rbase + (_RPC // _MCH) * _MCH, rem)],
                        hrow0.at[pl.ds(0, rem)])
        pltpu.sync_copy(hrow0.at[pl.ds(0, rem)],
                        aggr_hbm.at[pl.ds(obase + (_RPC // _MCH) * _MCH, rem)])


def _ehead_body(a_hbm, c_hbm, batch_hbm, row_hbm, col_hbm,
                ek_hbm, pk_hbm, pc_hbm,
                rowv, colv, av, cv, sgv, ekv,
                trowv, tcolv, tav, tcv, tsgv, tekv,
                acck, accc, foldv, sem1, sem2, sem3):
    cid = lax.axis_index("c")
    sid = lax.axis_index("s")
    wid = cid * _NS + sid
    zero = jnp.zeros((16,), jnp.float32)
    ones = jnp.ones((16,), jnp.float32)
    lanes = lax.iota(jnp.int32, 16)

    def zacc(i, _):
        acck[pl.ds(i * 16, 16)] = zero
        accc[pl.ds(i * 16, 16)] = zero
        return 0
    lax.fori_loop(0, 16 * _NG // 16, zacc, 0)

    ebase = wid * _EPT

    def do_group(src_a, src_c, src_sg, dst_ek, g):
        z = src_a[pl.ds(g * 16, 16)] + src_c[pl.ds(g * 16, 16)]
        ek = 1.0 / (1.0 + jnp.exp(-z))
        dst_ek[pl.ds(g * 16, 16)] = ek
        idx = lanes * _NG + src_sg[pl.ds(g * 16, 16)]
        plsc.addupdate_scatter(acck, [idx], ek)
        plsc.addupdate_scatter(accc, [idx], ones)

    def chunk(j, _):
        off = ebase + j * _CH
        pltpu.sync_copy(row_hbm.at[pl.ds(off, _CH)], rowv)
        pltpu.sync_copy(col_hbm.at[pl.ds(off, _CH)], colv)
        c1 = pltpu.async_copy(a_hbm.at[rowv], av, sem1)
        c2 = pltpu.async_copy(c_hbm.at[colv], cv, sem2)
        c3 = pltpu.async_copy(batch_hbm.at[rowv], sgv, sem3)
        c1.wait()
        c2.wait()
        c3.wait()
        for g in range(_CH // 16):
            do_group(av, cv, sgv, ekv, g)
        pltpu.sync_copy(ekv, ek_hbm.at[pl.ds(off, _CH)])
        return 0
    lax.fori_loop(0, _NFULL, chunk, 0)

    offt = ebase + _NFULL * _CH
    pltpu.sync_copy(row_hbm.at[pl.ds(offt, _TAIL)], trowv)
    pltpu.sync_copy(col_hbm.at[pl.ds(offt, _TAIL)], tcolv)
    c1 = pltpu.async_copy(a_hbm.at[trowv], tav, sem1)
    c2 = pltpu.async_copy(c_hbm.at[tcolv], tcv, sem2)
    c3 = pltpu.async_copy(batch_hbm.at[trowv], tsgv, sem3)
    c1.wait()
    c2.wait()
    c3.wait()
    do_group(tav, tcv, tsgv, tekv, 0)
    pltpu.sync_copy(tekv, ek_hbm.at[pl.ds(offt, _TAIL)])

    for acc, dst in ((acck, pk_hbm), (accc, pc_hbm)):
        def foldop(v, _):
            s = zero
            for l in range(16):
                s = s + acc[pl.ds(l * _NG + v * 16, 16)]
            foldv[pl.ds(v * 16, 16)] = s
            return 0
        lax.fori_loop(0, _NG // 16, foldop, 0)
        pltpu.sync_copy(foldv, dst.at[pl.ds(wid * _NG, _NG)])


@functools.lru_cache(maxsize=None)
def _sc_kernels():
    mesh = plsc.VectorSubcoreMesh(core_axis_name="c", subcore_axis_name="s",
                                  num_cores=_NC, num_subcores=_NS)
    msg = pl.kernel(
        _msg_body,
        out_type=jax.ShapeDtypeStruct((_NC * _NPAD, _D), jnp.float32),
        mesh=mesh,
        scratch_types=[
            pltpu.VMEM((3, _MCH), jnp.int32),
            pltpu.VMEM((_MCH, _D), jnp.float32),
            pltpu.VMEM((_MCH, _D), jnp.float32),
            pltpu.VMEM((3, _MCH), jnp.int32),
            pltpu.VMEM((_MCH, _D), jnp.float32),
            pltpu.VMEM((_MCH, _D), jnp.float32),
            pltpu.SemaphoreType.DMA,
            pltpu.SemaphoreType.DMA,
            pltpu.SemaphoreType.DMA,
            pltpu.SemaphoreType.DMA,
            pltpu.VMEM_SHARED((_NPAD, _D), jnp.float32),
        ],
    )

    ehead = pl.kernel(
        _ehead_body,
        out_type=(
            jax.ShapeDtypeStruct((_E,), jnp.float32),
            jax.ShapeDtypeStruct((_NT * _NG,), jnp.float32),
            jax.ShapeDtypeStruct((_NT * _NG,), jnp.float32),
        ),
        mesh=mesh,
        compiler_params=pltpu.CompilerParams(needs_layout_passes=False),
        scratch_types=[
            pltpu.VMEM((_CH,), jnp.int32),
            pltpu.VMEM((_CH,), jnp.int32),
            pltpu.VMEM((_CH,), jnp.float32),
            pltpu.VMEM((_CH,), jnp.float32),
            pltpu.VMEM((_CH,), jnp.int32),
            pltpu.VMEM((_CH,), jnp.float32),
            pltpu.VMEM((_TAIL,), jnp.int32),
            pltpu.VMEM((_TAIL,), jnp.int32),
            pltpu.VMEM((_TAIL,), jnp.float32),
            pltpu.VMEM((_TAIL,), jnp.float32),
            pltpu.VMEM((_TAIL,), jnp.int32),
            pltpu.VMEM((_TAIL,), jnp.float32),
            pltpu.VMEM((16 * _NG,), jnp.float32),
            pltpu.VMEM((16 * _NG,), jnp.float32),
            pltpu.VMEM((_NG,), jnp.float32),
            pltpu.SemaphoreType.DMA,
            pltpu.SemaphoreType.DMA,
            pltpu.SemaphoreType.DMA,
        ],
    )
    return msg, ehead

def _full(shape):
    nd = len(shape)
    return pl.BlockSpec(shape, lambda *ids: (0,) * nd)


def _build_tc(interpret=False):
    encode = pl.pallas_call(
        _encode_body,
        grid=(16,),
        in_specs=[
            pl.BlockSpec((_BR, 9), lambda i: (i, 0)),
            _full((9, 64, _D)),
            _full((16, _D)), _full((16, _D)), _full((16, _D)),
            _full((16, _D)), _full((16, _D)), _full((16, _D)),
        ],
        out_specs=(
            pl.BlockSpec((_BR, _D), lambda i: (i, 0)),
            _full((4096, _D)),
            _full((4096, _D)),
        ),
        out_shape=(
            jax.ShapeDtypeStruct((_NPAD, _D), jnp.float32),
            jax.ShapeDtypeStruct((4096, _D), jnp.float32),
            jax.ShapeDtypeStruct((4096, _D), jnp.float32),
        ),
        interpret=interpret,
    )

    def mlp(relu_out):
        return pl.pallas_call(
            functools.partial(_mlp_body, relu_out),
            grid=(3, 16),
            in_specs=[
                pl.BlockSpec((_BR, _D), lambda p, i: (i, 0)),
                pl.BlockSpec((2, _BR, _D), lambda p, i: (0, i, 0)),
                _full((1, 1)),
                _full((_D, 2 * _D)), _full((1, 2 * _D)),
                _full((1, 2 * _D)), _full((1, 2 * _D)),
                _full((2 * _D, _D)), _full((1, _D)),
                _full((1, _D)), _full((1, _D)),
            ],
            out_specs=pl.BlockSpec((_BR, _D), lambda p, i: (i, 0)),
            out_shape=jax.ShapeDtypeStruct((_NPAD, _D), jnp.float32),
            scratch_shapes=[
                pltpu.VMEM((_NPAD, 2 * _D), jnp.float32),
                pltpu.VMEM((_NPAD, _D), jnp.float32),
                pltpu.VMEM((8, 2 * _D), jnp.float32),
            ],
            interpret=interpret,
        )

    head = pl.pallas_call(
        _head_body,
        grid=(16,),
        in_specs=[
            pl.BlockSpec((_BR, _D), lambda i: (i, 0)),
            pl.BlockSpec((1, 1, _BR), lambda i: (i, 0, 0)),
            _full((_D, _D)),
            _full((1, _D)),
        ],
        out_specs=(
            pl.BlockSpec((_BR, _D), lambda i: (i, 0)),
            _full((_NG, _D)),
        ),
        out_shape=(
            jax.ShapeDtypeStruct((_NPAD, _D), jnp.float32),
            jax.ShapeDtypeStruct((_NG, _D), jnp.float32),
        ),
        interpret=interpret,
    )

    fin = pl.pallas_call(
        _fin_body,
        out_shape=jax.ShapeDtypeStruct((2, _NG), jnp.float32),
        interpret=interpret,
    )
    return encode, mlp(True), mlp(False), head, fin


_encode, _mlp0, _mlp1, _head, _fin = _build_tc()


def kernel(x, edge_index, edge_attr, batch, params):
    row = edge_index[0]
    col = edge_index[1]
    pidx = edge_attr[:, 0] * 256 + edge_attr[:, 1] * 16 + edge_attr[:, 2]
    pk = jnp.stack([row.reshape(_NCH, _MCH), col.reshape(_NCH, _MCH),
                    pidx.reshape(_NCH, _MCH)], axis=1)
    l0, l1 = params['layers']
    _msg, _ehead = _sc_kernels()

    h0, t0, t1 = _encode(x, params['atom_emb'],
                         l0['bond_emb'][0], l0['bond_emb'][1],
                         l0['bond_emb'][2],
                         l1['bond_emb'][0], l1['bond_emb'][1],
                         l1['bond_emb'][2])
    ag0 = _msg(h0, t0, pk)
    eps0 = l0['eps'].reshape(1, 1)
    h1 = _mlp0(h0, ag0.reshape(_NC, _NPAD, _D), eps0,
               l0['W1'], l0['b1'].reshape(1, -1),
               l0['bn1_g'].reshape(1, -1), l0['bn1_b'].reshape(1, -1),
               l0['W2'], l0['b2'].reshape(1, -1),
               l0['bno_g'].reshape(1, -1), l0['bno_b'].reshape(1, -1))
    ag1 = _msg(h1, t1, pk)
    eps1 = l1['eps'].reshape(1, 1)
    h2 = _mlp1(h1, ag1.reshape(_NC, _NPAD, _D), eps1,
               l1['W1'], l1['b1'].reshape(1, -1),
               l1['bn1_g'].reshape(1, -1), l1['bn1_b'].reshape(1, -1),
               l1['W2'], l1['b2'].reshape(1, -1),
               l1['bno_g'].reshape(1, -1), l1['bno_b'].reshape(1, -1))

    wcat = jnp.zeros((_D, _D), jnp.float32)
    wcat = wcat.at[:, 0].set(params['node_att_W'][:, 0])
    wcat = wcat.at[:, 1].set(params['edge_att_W'][:_D, 0])
    wcat = wcat.at[:, 2].set(params['edge_att_W'][_D:, 0])
    bvec = jnp.zeros((1, _D), jnp.float32)
    bvec = bvec.at[0, 0].set(params['node_att_b'][0])
    bvec = bvec.at[0, 1].set(params['edge_att_b'][0])

    batch_pad = jnp.concatenate(
        [batch, jnp.zeros((_NPAD - _N,), jnp.int32)]).reshape(16, 1, _BR)
    p, r = _head(h2, batch_pad, wcat, bvec)
    p = p[:_N]
    a = p[:, 1]
    c = p[:, 2]
    ek, pk, pc = _ehead(a, c, batch, row, col)
    fin = _fin(pk.reshape(_NT, _NG), pc.reshape(_NT, _NG))

    node_key = p[:, 0:1]
    edge_key = ek.reshape(_E, 1)
    node_key_num = r[:, 0:1]
    node_env_num = r[:, 1:2]
    edge_key_num = fin[0].reshape(_NG, 1)
    edge_env_num = fin[1].reshape(_NG, 1)
    return (node_key, edge_key, node_key_num, node_env_num,
            edge_key_num, edge_env_num)
